# baseline (speedup 1.0000x reference)
import functools

import jax
import jax.numpy as jnp
from jax import lax
from jax.experimental import pallas as pl
from jax.experimental.pallas import tpu as pltpu
from jax.experimental.pallas import tpu_sc as plsc

D3 = 128
BATCH = 16384
NC = 2
NS = 16
NW = NC * NS
BPW = BATCH // NW
CH = 32
NCH = BPW // CH
NBUF = 8
SLAB = 256 // NS

_mesh = plsc.VectorSubcoreMesh(core_axis_name="c", subcore_axis_name="s")


@functools.partial(
    pl.kernel,
    mesh=_mesh,
    out_type=jax.ShapeDtypeStruct((BATCH, 3 * D3), jnp.float32),
    scratch_types=[
        pltpu.VMEM_SHARED((256, D3), jnp.float32),
        pltpu.VMEM_SHARED((256, D3), jnp.float32),
        pltpu.VMEM_SHARED((256, D3), jnp.float32),
        pltpu.VMEM((BPW,), jnp.int32),
        pltpu.VMEM((BPW,), jnp.int32),
        pltpu.VMEM((BPW,), jnp.int32),
        pltpu.VMEM((NBUF, CH, D3), jnp.float32),
        pltpu.VMEM((NBUF, CH, D3), jnp.float32),
        pltpu.VMEM((NBUF, CH, D3), jnp.float32),
        pltpu.SemaphoreType.DMA,
        pltpu.SemaphoreType.DMA,
        pltpu.SemaphoreType.DMA,
        pltpu.SemaphoreType.DMA,
        pltpu.SemaphoreType.DMA,
        pltpu.SemaphoreType.DMA,
        pltpu.SemaphoreType.DMA,
        pltpu.SemaphoreType.DMA,
        pltpu.SemaphoreType.DMA,
        pltpu.SemaphoreType.DMA,
        pltpu.SemaphoreType.DMA,
        pltpu.SemaphoreType.DMA,
        pltpu.SemaphoreType.DMA,
        pltpu.SemaphoreType.DMA,
        pltpu.SemaphoreType.DMA,
        pltpu.SemaphoreType.DMA,
        pltpu.SemaphoreType.DMA,
    ],
)
def _pe3d(xh, yh, zh, xt, yt, zt, out, spx, spy, spz, xi, yi, zi,
          rx, ry, rz, g0, g1, g2, g3, g4, g5, g6, g7,
          w0, w1, w2, w3, w4, w5, w6, w7, ssem):
    gsems = (g0, g1, g2, g3, g4, g5, g6, g7)
    wsems = (w0, w1, w2, w3, w4, w5, w6, w7)
    sid = lax.axis_index("s")
    wid = sid * NC + lax.axis_index("c")
    base = wid * BPW

    slab = pl.ds(sid * SLAB, SLAB)
    s1 = pltpu.async_copy(xt.at[slab], spx.at[slab], ssem)
    s2 = pltpu.async_copy(yt.at[slab], spy.at[slab], ssem)
    s3 = pltpu.async_copy(zt.at[slab], spz.at[slab], ssem)
    i1 = pltpu.async_copy(xh.at[pl.ds(base, BPW)], xi, ssem)
    i2 = pltpu.async_copy(yh.at[pl.ds(base, BPW)], yi, ssem)
    i3 = pltpu.async_copy(zh.at[pl.ds(base, BPW)], zi, ssem)
    i1.wait()
    i2.wait()
    i3.wait()
    s1.wait()
    s2.wait()
    s3.wait()
    plsc.subcore_barrier()

    gathers = [None] * NCH
    writes = [None] * NCH

    def issue_writes(ci):
        for g in gathers[ci]:
            g.wait()
        b = ci % NBUF
        r0 = base + ci * CH
        ws = wsems[b]
        writes[ci] = (
            pltpu.async_copy(rx.at[b], out.at[pl.ds(r0, CH), pl.ds(0, D3)], ws),
            pltpu.async_copy(ry.at[b], out.at[pl.ds(r0, CH), pl.ds(D3, D3)], ws),
            pltpu.async_copy(rz.at[b], out.at[pl.ds(r0, CH), pl.ds(2 * D3, D3)], ws),
        )

    for ci in range(NCH):
        b = ci % NBUF
        if ci >= NBUF:
            for w in writes[ci - NBUF]:
                w.wait()
        sl = pl.ds(ci * CH, CH)
        gs = gsems[b]
        gathers[ci] = (
            pltpu.async_copy(spx.at[xi.at[sl]], rx.at[b], gs),
            pltpu.async_copy(spy.at[yi.at[sl]], ry.at[b], gs),
            pltpu.async_copy(spz.at[zi.at[sl]], rz.at[b], gs),
        )
        if ci >= 1:
            issue_writes(ci - 1)
    issue_writes(NCH - 1)
    for ci in range(NCH - NBUF, NCH):
        for w in writes[ci]:
            w.wait()


def kernel(x, y, z, x_pos, y_pos, z_pos):
    return _pe3d(
        x.astype(jnp.int32),
        y.astype(jnp.int32),
        z.astype(jnp.int32),
        x_pos,
        y_pos,
        z_pos,
    )


# speedup vs baseline: 1.0321x; 1.0321x over previous
import functools

import jax
import jax.numpy as jnp
from jax import lax
from jax.experimental import pallas as pl
from jax.experimental.pallas import tpu as pltpu
from jax.experimental.pallas import tpu_sc as plsc

D3 = 128
BATCH = 16384
NC = 2
NS = 16
NW = NC * NS
BPW = BATCH // NW
CH = 64
NCH = BPW // CH
NBUF = 4
SLAB = 256 // NS

_mesh = plsc.VectorSubcoreMesh(core_axis_name="c", subcore_axis_name="s")


@functools.partial(
    pl.kernel,
    mesh=_mesh,
    out_type=jax.ShapeDtypeStruct((BATCH, 3 * D3), jnp.float32),
    scratch_types=[
        pltpu.VMEM_SHARED((256, D3), jnp.float32),
        pltpu.VMEM_SHARED((256, D3), jnp.float32),
        pltpu.VMEM_SHARED((256, D3), jnp.float32),
        pltpu.VMEM((BPW,), jnp.int32),
        pltpu.VMEM((BPW,), jnp.int32),
        pltpu.VMEM((BPW,), jnp.int32),
        pltpu.VMEM((NBUF, CH, D3), jnp.float32),
        pltpu.VMEM((NBUF, CH, D3), jnp.float32),
        pltpu.VMEM((NBUF, CH, D3), jnp.float32),
        pltpu.SemaphoreType.DMA,
        pltpu.SemaphoreType.DMA,
        pltpu.SemaphoreType.DMA,
        pltpu.SemaphoreType.DMA,
        pltpu.SemaphoreType.DMA,
        pltpu.SemaphoreType.DMA,
        pltpu.SemaphoreType.DMA,
        pltpu.SemaphoreType.DMA,
        pltpu.SemaphoreType.DMA,
    ],
)
def _pe3d(xh, yh, zh, xt, yt, zt, out, spx, spy, spz, xi, yi, zi,
          rx, ry, rz, g0, g1, g2, g3, w0, w1, w2, w3, ssem):
    gsems = (g0, g1, g2, g3)
    wsems = (w0, w1, w2, w3)
    sid = lax.axis_index("s")
    wid = sid * NC + lax.axis_index("c")
    base = wid * BPW

    slab = pl.ds(sid * SLAB, SLAB)
    s1 = pltpu.async_copy(xt.at[slab], spx.at[slab], ssem)
    s2 = pltpu.async_copy(yt.at[slab], spy.at[slab], ssem)
    s3 = pltpu.async_copy(zt.at[slab], spz.at[slab], ssem)
    i1 = pltpu.async_copy(xh.at[pl.ds(base, BPW)], xi, ssem)
    i2 = pltpu.async_copy(yh.at[pl.ds(base, BPW)], yi, ssem)
    i3 = pltpu.async_copy(zh.at[pl.ds(base, BPW)], zi, ssem)
    i1.wait()
    i2.wait()
    i3.wait()
    s1.wait()
    s2.wait()
    s3.wait()
    plsc.subcore_barrier()

    gathers = [None] * NCH
    writes = [None] * NCH

    def issue_writes(ci):
        for g in gathers[ci]:
            g.wait()
        b = ci % NBUF
        r0 = base + ci * CH
        ws = wsems[b]
        writes[ci] = (
            pltpu.async_copy(rx.at[b], out.at[pl.ds(r0, CH), pl.ds(0, D3)], ws),
            pltpu.async_copy(ry.at[b], out.at[pl.ds(r0, CH), pl.ds(D3, D3)], ws),
            pltpu.async_copy(rz.at[b], out.at[pl.ds(r0, CH), pl.ds(2 * D3, D3)], ws),
        )

    for ci in range(NCH):
        b = ci % NBUF
        if ci >= NBUF:
            for w in writes[ci - NBUF]:
                w.wait()
        sl = pl.ds(ci * CH, CH)
        gs = gsems[b]
        gathers[ci] = (
            pltpu.async_copy(spx.at[xi.at[sl]], rx.at[b], gs),
            pltpu.async_copy(spy.at[yi.at[sl]], ry.at[b], gs),
            pltpu.async_copy(spz.at[zi.at[sl]], rz.at[b], gs),
        )
        if ci >= 1:
            issue_writes(ci - 1)
    issue_writes(NCH - 1)
    for ci in range(NCH - NBUF, NCH):
        for w in writes[ci]:
            w.wait()


def kernel(x, y, z, x_pos, y_pos, z_pos):
    return _pe3d(
        x.astype(jnp.int32),
        y.astype(jnp.int32),
        z.astype(jnp.int32),
        x_pos,
        y_pos,
        z_pos,
    )


# speedup vs baseline: 1.0353x; 1.0031x over previous
import functools

import jax
import jax.numpy as jnp
from jax import lax
from jax.experimental import pallas as pl
from jax.experimental.pallas import tpu as pltpu
from jax.experimental.pallas import tpu_sc as plsc

D3 = 128
BATCH = 16384
NC = 2
NS = 16
NW = NC * NS
BPW = BATCH // NW
CH = 64
NCH = BPW // CH
NBUF = 4
SLAB = 256 // NS

_mesh = plsc.VectorSubcoreMesh(core_axis_name="c", subcore_axis_name="s")


@functools.partial(
    pl.kernel,
    mesh=_mesh,
    out_type=jax.ShapeDtypeStruct((BATCH, 3 * D3), jnp.float32),
    scratch_types=[
        pltpu.VMEM_SHARED((256, D3), jnp.float32),
        pltpu.VMEM_SHARED((256, D3), jnp.float32),
        pltpu.VMEM_SHARED((256, D3), jnp.float32),
        pltpu.VMEM((BPW,), jnp.int32),
        pltpu.VMEM((BPW,), jnp.int32),
        pltpu.VMEM((BPW,), jnp.int32),
        pltpu.VMEM((NBUF, CH, D3), jnp.float32),
        pltpu.VMEM((NBUF, CH, D3), jnp.float32),
        pltpu.VMEM((NBUF, CH, D3), jnp.float32),
        pltpu.SemaphoreType.DMA,
        pltpu.SemaphoreType.DMA,
        pltpu.SemaphoreType.DMA,
        pltpu.SemaphoreType.DMA,
        pltpu.SemaphoreType.DMA,
        pltpu.SemaphoreType.DMA,
        pltpu.SemaphoreType.DMA,
        pltpu.SemaphoreType.DMA,
        pltpu.SemaphoreType.DMA,
    ],
)
def _pe3d(xh, yh, zh, xt, yt, zt, out, spx, spy, spz, xi, yi, zi,
          rx, ry, rz, g0, g1, g2, g3, w0, w1, w2, w3, ssem):
    gsems = (g0, g1, g2, g3)
    wsems = (w0, w1, w2, w3)
    sid = lax.axis_index("s")
    wid = sid * NC + lax.axis_index("c")
    base = wid * BPW

    slab = pl.ds(sid * SLAB, SLAB)
    s1 = pltpu.async_copy(xt.at[slab], spx.at[slab], ssem)
    s2 = pltpu.async_copy(yt.at[slab], spy.at[slab], ssem)
    s3 = pltpu.async_copy(zt.at[slab], spz.at[slab], ssem)
    i1 = pltpu.async_copy(xh.at[pl.ds(base, BPW)], xi, ssem)
    i2 = pltpu.async_copy(yh.at[pl.ds(base, BPW)], yi, ssem)
    i3 = pltpu.async_copy(zh.at[pl.ds(base, BPW)], zi, ssem)
    i1.wait()
    i2.wait()
    i3.wait()
    s1.wait()
    s2.wait()
    s3.wait()
    plsc.subcore_barrier()

    gathers = [None] * NCH
    writes = [None] * NCH

    def issue_writes(ci):
        for g in gathers[ci]:
            g.wait()
        b = ci % NBUF
        r0 = base + ci * CH
        ws = wsems[b]
        writes[ci] = (
            pltpu.async_copy(rx.at[b], out.at[pl.ds(r0, CH), pl.ds(0, D3)], ws),
            pltpu.async_copy(ry.at[b], out.at[pl.ds(r0, CH), pl.ds(D3, D3)], ws),
            pltpu.async_copy(rz.at[b], out.at[pl.ds(r0, CH), pl.ds(2 * D3, D3)], ws),
        )

    for ci in range(NCH):
        b = ci % NBUF
        if ci >= NBUF:
            for w in writes[ci - NBUF]:
                w.wait()
        sl = pl.ds(ci * CH, CH)
        gs = gsems[b]
        gathers[ci] = (
            pltpu.async_copy(spx.at[xi.at[sl]], rx.at[b], gs),
            pltpu.async_copy(spy.at[yi.at[sl]], ry.at[b], gs),
            pltpu.async_copy(spz.at[zi.at[sl]], rz.at[b], gs),
        )
        if ci >= 2:
            issue_writes(ci - 2)
    issue_writes(NCH - 2)
    issue_writes(NCH - 1)
    for ci in range(NCH - NBUF, NCH):
        for w in writes[ci]:
            w.wait()


def kernel(x, y, z, x_pos, y_pos, z_pos):
    return _pe3d(
        x.astype(jnp.int32),
        y.astype(jnp.int32),
        z.astype(jnp.int32),
        x_pos,
        y_pos,
        z_pos,
    )


# speedup vs baseline: 1.0462x; 1.0106x over previous
import functools

import jax
import jax.numpy as jnp
from jax import lax
from jax.experimental import pallas as pl
from jax.experimental.pallas import tpu as pltpu
from jax.experimental.pallas import tpu_sc as plsc

D3 = 128
BATCH = 16384
NC = 2
NS = 16
NW = NC * NS
BPW = BATCH // NW
CH = 64
NCH = BPW // CH
NBUF = 4
SLAB = 256 // NS

_mesh = plsc.VectorSubcoreMesh(core_axis_name="c", subcore_axis_name="s")


@functools.partial(
    pl.kernel,
    mesh=_mesh,
    compiler_params=pltpu.CompilerParams(
        disable_bounds_checks=True,
        disable_semaphore_checks=True,
    ),
    out_type=jax.ShapeDtypeStruct((BATCH, 3 * D3), jnp.float32),
    scratch_types=[
        pltpu.VMEM_SHARED((256, D3), jnp.float32),
        pltpu.VMEM_SHARED((256, D3), jnp.float32),
        pltpu.VMEM_SHARED((256, D3), jnp.float32),
        pltpu.VMEM((BPW,), jnp.int32),
        pltpu.VMEM((BPW,), jnp.int32),
        pltpu.VMEM((BPW,), jnp.int32),
        pltpu.VMEM((NBUF, CH, D3), jnp.float32),
        pltpu.VMEM((NBUF, CH, D3), jnp.float32),
        pltpu.VMEM((NBUF, CH, D3), jnp.float32),
        pltpu.SemaphoreType.DMA,
        pltpu.SemaphoreType.DMA,
        pltpu.SemaphoreType.DMA,
        pltpu.SemaphoreType.DMA,
        pltpu.SemaphoreType.DMA,
        pltpu.SemaphoreType.DMA,
        pltpu.SemaphoreType.DMA,
        pltpu.SemaphoreType.DMA,
        pltpu.SemaphoreType.DMA,
    ],
)
def _pe3d(xh, yh, zh, xt, yt, zt, out, spx, spy, spz, xi, yi, zi,
          rx, ry, rz, g0, g1, g2, g3, w0, w1, w2, w3, ssem):
    gsems = (g0, g1, g2, g3)
    wsems = (w0, w1, w2, w3)
    sid = lax.axis_index("s")
    wid = sid * NC + lax.axis_index("c")
    base = wid * BPW

    slab = pl.ds(sid * SLAB, SLAB)
    s1 = pltpu.async_copy(xt.at[slab], spx.at[slab], ssem)
    s2 = pltpu.async_copy(yt.at[slab], spy.at[slab], ssem)
    s3 = pltpu.async_copy(zt.at[slab], spz.at[slab], ssem)
    i1 = pltpu.async_copy(xh.at[pl.ds(base, BPW)], xi, ssem)
    i2 = pltpu.async_copy(yh.at[pl.ds(base, BPW)], yi, ssem)
    i3 = pltpu.async_copy(zh.at[pl.ds(base, BPW)], zi, ssem)
    i1.wait()
    i2.wait()
    i3.wait()
    s1.wait()
    s2.wait()
    s3.wait()
    plsc.subcore_barrier()

    gathers = [None] * NCH
    writes = [None] * NCH

    def issue_writes(ci):
        for g in gathers[ci]:
            g.wait()
        b = ci % NBUF
        r0 = base + ci * CH
        ws = wsems[b]
        writes[ci] = (
            pltpu.async_copy(rx.at[b], out.at[pl.ds(r0, CH), pl.ds(0, D3)], ws),
            pltpu.async_copy(ry.at[b], out.at[pl.ds(r0, CH), pl.ds(D3, D3)], ws),
            pltpu.async_copy(rz.at[b], out.at[pl.ds(r0, CH), pl.ds(2 * D3, D3)], ws),
        )

    for ci in range(NCH):
        b = ci % NBUF
        if ci >= NBUF:
            for w in writes[ci - NBUF]:
                w.wait()
        sl = pl.ds(ci * CH, CH)
        gs = gsems[b]
        gathers[ci] = (
            pltpu.async_copy(spx.at[xi.at[sl]], rx.at[b], gs),
            pltpu.async_copy(spy.at[yi.at[sl]], ry.at[b], gs),
            pltpu.async_copy(spz.at[zi.at[sl]], rz.at[b], gs),
        )
        if ci >= 2:
            issue_writes(ci - 2)
    issue_writes(NCH - 2)
    issue_writes(NCH - 1)
    for ci in range(NCH - NBUF, NCH):
        for w in writes[ci]:
            w.wait()


def kernel(x, y, z, x_pos, y_pos, z_pos):
    return _pe3d(
        x.astype(jnp.int32),
        y.astype(jnp.int32),
        z.astype(jnp.int32),
        x_pos,
        y_pos,
        z_pos,
    )
